# Initial kernel scaffold; baseline (speedup 1.0000x reference)
import jax
import jax.numpy as jnp
from jax.experimental import pallas as pl


def kernel(q, k, v, attention_mask):
    raise NotImplementedError("write your pallas kernel here")


# speedup vs baseline: 58.2877x; 58.2877x over previous
import functools

import jax
import jax.numpy as jnp
from jax.experimental import pallas as pl

_SPARQ = 32
_TOP_K = 512


def _ordered_u32(x):
    u = jax.lax.bitcast_convert_type(x, jnp.uint32)
    flip = jnp.where(u >= jnp.uint32(0x80000000),
                     jnp.uint32(0xFFFFFFFF), jnp.uint32(0x80000000))
    return u ^ flip


def _kth_largest_u32(o, kk, nbits=32):
    R = o.shape[0]
    p = jnp.zeros((R, 1), jnp.uint32)
    for b in range(nbits - 1, -1, -1):
        cand = p | jnp.uint32(1 << b)
        cnt = jnp.sum((o >= cand).astype(jnp.int32), axis=1, keepdims=True)
        p = jnp.where(cnt >= kk, cand, p)
    return p


def _body(q_ref, k_ref, v_ref, am_ref, o_ref, *, R, T, HID, kk):
    qt = q_ref[0]
    kt = k_ref[0]
    vt = v_ref[0]
    am = am_ref[...]

    aq = jnp.abs(qt)
    ai = jax.lax.bitcast_convert_type(aq, jnp.uint32)
    pq = _kth_largest_u32(ai, _SPARQ, nbits=31)
    qs = jnp.where(ai >= pq, qt, jnp.float32(0.0))

    approx = jax.lax.dot_general(qs, kt, (((1,), (1,)), ((), ())),
                                 preferred_element_type=jnp.float32)
    rows = pl.program_id(1) * R + jax.lax.broadcasted_iota(jnp.int32, (R, T), 0)
    cols = jax.lax.broadcasted_iota(jnp.int32, (R, T), 1)
    valid = (cols <= rows) & (am > jnp.float32(0.5))
    ax = jnp.where(valid, approx, jnp.float32(-jnp.inf))

    o = _ordered_u32(ax)
    pth = _kth_largest_u32(o, kk)
    sel = (o >= pth) & valid

    scores = jax.lax.dot_general(qt, kt, (((1,), (1,)), ((), ())),
                                 preferred_element_type=jnp.float32)
    s = jnp.where(sel, scores, jnp.float32(-32000.0))
    m = jnp.max(s, axis=1, keepdims=True)
    e = jnp.exp(s - m)
    num = jax.lax.dot_general(e, vt, (((1,), (0,)), ((), ())),
                              preferred_element_type=jnp.float32)
    denom = jnp.sum(e, axis=1, keepdims=True)
    o_ref[0] = num / denom


def kernel(q, k, v, attention_mask):
    N, H, T, HID = q.shape
    B = N * H
    R = min(256, T)
    kk = min(_TOP_K, T)
    qf = q.reshape(B, T, HID)
    kf = k.reshape(B, T, HID)
    vf = v.reshape(B, T, HID)

    out = pl.pallas_call(
        functools.partial(_body, R=R, T=T, HID=HID, kk=kk),
        grid=(B, T // R),
        in_specs=[
            pl.BlockSpec((1, R, HID), lambda b, t: (b, t, 0)),
            pl.BlockSpec((1, T, HID), lambda b, t: (b, 0, 0)),
            pl.BlockSpec((1, T, HID), lambda b, t: (b, 0, 0)),
            pl.BlockSpec((1, T), lambda b, t: (b // H, 0)),
        ],
        out_specs=pl.BlockSpec((1, R, HID), lambda b, t: (b, t, 0)),
        out_shape=jax.ShapeDtypeStruct((B, T, HID), jnp.float32),
    )(qf, kf, vf, attention_mask)
    return out.reshape(N, H, T, HID)


# speedup vs baseline: 91.6868x; 1.5730x over previous
import functools

import jax
import jax.numpy as jnp
from jax.experimental import pallas as pl

_SPARQ = 32
_TOP_K = 512


def _ordered_u32(x):
    u = jax.lax.bitcast_convert_type(x, jnp.uint32)
    flip = jnp.where(u >= jnp.uint32(0x80000000),
                     jnp.uint32(0xFFFFFFFF), jnp.uint32(0x80000000))
    return u ^ flip


def _kth_largest_u32(o, kk, nbits=32):
    R = o.shape[0]
    p = jnp.zeros((R, 1), jnp.uint32)
    for b in range(nbits - 1, -1, -1):
        cand = p | jnp.uint32(1 << b)
        cnt = jnp.sum((o >= cand).astype(jnp.int32), axis=1, keepdims=True)
        p = jnp.where(cnt >= kk, cand, p)
    return p


def _body(q_ref, k_ref, v_ref, am_ref, o_ref, *, R, E, kk, t0, do_select):
    qt = q_ref[0]
    kt = k_ref[0]
    vt = v_ref[0]
    am = am_ref[...]

    rows = t0 + jax.lax.broadcasted_iota(jnp.int32, (R, E), 0)
    cols = jax.lax.broadcasted_iota(jnp.int32, (R, E), 1)
    valid = (cols <= rows) & (am > jnp.float32(0.5))

    scores = jax.lax.dot_general(qt, kt, (((1,), (1,)), ((), ())),
                                 preferred_element_type=jnp.float32)

    if do_select:
        aq = jnp.abs(qt)
        ai = jax.lax.bitcast_convert_type(aq, jnp.uint32)
        pq = _kth_largest_u32(ai, _SPARQ, nbits=31)
        qs = jnp.where(ai >= pq, qt, jnp.float32(0.0))

        approx = jax.lax.dot_general(qs, kt, (((1,), (1,)), ((), ())),
                                     preferred_element_type=jnp.float32)
        ax = jnp.where(valid, approx, jnp.float32(-jnp.inf))

        o = _ordered_u32(ax)
        pth = _kth_largest_u32(o, kk)
        sel = (o >= pth) & valid
    else:
        sel = valid

    s = jnp.where(sel, scores, jnp.float32(-32000.0))
    m = jnp.max(s, axis=1, keepdims=True)
    e = jnp.exp(s - m)
    num = jax.lax.dot_general(e, vt, (((1,), (0,)), ((), ())),
                              preferred_element_type=jnp.float32)
    denom = jnp.sum(e, axis=1, keepdims=True)
    o_ref[0] = num / denom


def kernel(q, k, v, attention_mask):
    N, H, T, HID = q.shape
    B = N * H
    R = min(256, T)
    nt = T // R
    kk = min(_TOP_K, T)
    qf = q.reshape(B, T, HID)
    kf = k.reshape(B, T, HID)
    vf = v.reshape(B, T, HID)

    pieces = []
    for ti in range(nt):
        E = (ti + 1) * R
        pieces.append(pl.pallas_call(
            functools.partial(_body, R=R, E=E, kk=kk, t0=ti * R,
                              do_select=(E > kk)),
            grid=(B,),
            in_specs=[
                pl.BlockSpec((1, R, HID), lambda b, ti=ti: (b, ti, 0)),
                pl.BlockSpec((1, E, HID), lambda b: (b, 0, 0)),
                pl.BlockSpec((1, E, HID), lambda b: (b, 0, 0)),
                pl.BlockSpec((1, E), lambda b: (b // H, 0)),
            ],
            out_specs=pl.BlockSpec((1, R, HID), lambda b: (b, 0, 0)),
            out_shape=jax.ShapeDtypeStruct((B, R, HID), jnp.float32),
        )(qf, kf, vf, attention_mask))
    out = jnp.concatenate(pieces, axis=1)
    return out.reshape(N, H, T, HID)
